# baseline (speedup 1.0000x reference)
import jax
import jax.numpy as jnp
from jax.experimental import pallas as pl
from jax.experimental.pallas import tpu as pltpu

_BLK = 512


def _scale_body(hw_ref, sim_ref, lm_ref, out_ref, pout_ref):
    e = pl.program_id(0)
    ne = sim_ref.shape[1]
    sim = sim_ref[...]
    idx = jax.lax.broadcasted_iota(jnp.int32, (1, ne), 1)
    total = jnp.sum(sim)
    s0 = jnp.sum(jnp.where(idx == 0, sim, 0.0))
    mean_rest = (total - s0) / (ne - 1)
    adj = jnp.where(idx == 0, mean_rest, sim)
    m = jnp.max(adj)
    ex = jnp.exp(adj - m)
    soft = ex / jnp.sum(ex)
    amax = jnp.min(jnp.where(adj == m, idx, ne))
    onehot = (idx == amax).astype(jnp.float32)
    hw = hw_ref[0]
    pvec = jnp.where(hw > 0.5, onehot, soft)
    s = jnp.sum(jnp.where(idx == e, pvec, 0.0))
    out_ref[...] = lm_ref[...] * s
    pout_ref[...] = jnp.full(pout_ref.shape, s, jnp.float32)


def kernel(lm_logits, encoder_hidden, decoder_hidden, n_expert, similarity, hard_weight):
    del encoder_hidden, decoder_hidden, n_expert
    ne, Ld, vocab = lm_logits.shape
    sim2 = similarity.astype(jnp.float32).reshape(1, ne)
    hw = jnp.asarray(hard_weight, jnp.float32).reshape(1)
    nblk = Ld // _BLK
    out, p = pl.pallas_call(
        _scale_body,
        grid=(ne, nblk),
        in_specs=[
            pl.BlockSpec(memory_space=pltpu.SMEM),
            pl.BlockSpec((1, ne), lambda e, i: (0, 0)),
            pl.BlockSpec((1, _BLK, vocab), lambda e, i: (e, i, 0)),
        ],
        out_specs=[
            pl.BlockSpec((1, _BLK, vocab), lambda e, i: (e, i, 0)),
            pl.BlockSpec((1, _BLK, 1), lambda e, i: (e, i, 0)),
        ],
        out_shape=[
            jax.ShapeDtypeStruct((ne, Ld, vocab), jnp.float32),
            jax.ShapeDtypeStruct((ne, Ld, 1), jnp.float32),
        ],
        compiler_params=pltpu.CompilerParams(
            dimension_semantics=("parallel", "parallel"),
        ),
    )(hw, sim2, lm_logits)
    return (out, p)
